# baseline (speedup 1.0000x reference)
import functools

import jax
import jax.numpy as jnp
from jax import lax
from jax.experimental import pallas as pl
from jax.experimental.pallas import tpu as pltpu
from jax.experimental.pallas import tpu_sc as plsc

N = 10000
NPAD = 10240
E = 320000
D = 128
DH = D // 2
NC, NS = 2, 16
NW = NC * NS
CH = 80
EPT = E // NW
NCHUNK = EPT // CH
NBUF = 10
LOOK = 9
STRIPE = NPAD // NS
RB = 1024
NP2 = NPAD // 2
RBP = 512

_sc_mesh = plsc.VectorSubcoreMesh(
    core_axis_name="c", subcore_axis_name="s", num_cores=NC, num_subcores=NS)


@functools.partial(
    pl.kernel,
    out_type=jax.ShapeDtypeStruct((NC, NP2, D), jnp.float32),
    mesh=_sc_mesh,
    scratch_types=[
        pltpu.VMEM((EPT // 80, 80), jnp.int32),
        pltpu.VMEM((80,), jnp.float32),
        pltpu.VMEM((STRIPE,), jnp.float32),
        pltpu.VMEM((STRIPE // 2, D), jnp.float32),
        pltpu.VMEM_SHARED((NPAD,), jnp.float32),
    ],
    compiler_params=pltpu.CompilerParams(needs_layout_passes=False),
)
def _deg_kernel(ei_hbm, out_hbm, idx_v, ones_v, zero_v, bb, acc):
    c = lax.axis_index("c")
    s = lax.axis_index("s")
    wid = c * NS + s
    nch = EPT // 80

    for k in range(80 // 16):
        ones_v[pl.ds(k * 16, 16)] = jnp.ones((16,), jnp.float32)

    def _zf(i, carry):
        zero_v[pl.ds(i * 16, 16)] = jnp.zeros((16,), jnp.float32)
        return carry
    lax.fori_loop(0, STRIPE // 16, _zf, 0)

    pltpu.sync_copy(ei_hbm.at[1, wid], idx_v)
    pltpu.sync_copy(zero_v, acc.at[pl.ds(s * STRIPE, STRIPE)])
    plsc.subcore_barrier()

    def _body(j, carry):
        pltpu.sync_copy(ones_v, acc.at[idx_v.at[j]], add=True)
        return carry
    lax.fori_loop(0, nch, _body, 0)

    plsc.subcore_barrier()
    pltpu.sync_copy(acc.at[pl.ds(s * STRIPE, STRIPE)], zero_v)

    def _bc(r, carry):
        v0 = plsc.load_gather(zero_v, [jnp.full((16,), 2 * r, jnp.int32)])
        v1 = plsc.load_gather(zero_v, [jnp.full((16,), 2 * r + 1, jnp.int32)])
        for q in range(DH // 16):
            bb[r, pl.ds(q * 16, 16)] = v0
        for q in range(DH // 16):
            bb[r, pl.ds(DH + q * 16, 16)] = v1
        return carry
    lax.fori_loop(0, STRIPE // 2, _bc, 0)
    pltpu.sync_copy(bb, out_hbm.at[c, pl.ds(s * (STRIPE // 2), STRIPE // 2)])


@functools.partial(
    pl.kernel,
    out_type=jax.ShapeDtypeStruct((NC, 2, NPAD, DH), jnp.float32),
    mesh=_sc_mesh,
    scratch_types=[
        pltpu.VMEM((NCHUNK, CH), jnp.int32),
        pltpu.VMEM((NCHUNK, CH), jnp.int32),
        [pltpu.VMEM((CH, DH), jnp.float32) for _ in range(NBUF)],
        pltpu.VMEM_SHARED((NPAD, DH), jnp.float32),
        [pltpu.SemaphoreType.DMA for _ in range(NBUF)],
        [pltpu.SemaphoreType.DMA for _ in range(NBUF)],
    ],
    compiler_params=pltpu.CompilerParams(use_tc_tiling_on_sc=False),
)
def _scatter_kernel(g_lo_hbm, g_hi_hbm, ei_hbm, out_hbm,
                    src_v, dst_v, bufs, acc, gsems, ssems):
    c = lax.axis_index("c")
    s = lax.axis_index("s")
    wid = c * NS + s

    pltpu.sync_copy(ei_hbm.at[0, wid], src_v)
    pltpu.sync_copy(ei_hbm.at[1, wid], dst_v)

    for f, g_hbm in ((0, g_lo_hbm), (1, g_hi_hbm)):
        def _zf(i, carry):
            for k in range(DH // 16):
                bufs[0][i, pl.ds(k * 16, 16)] = jnp.zeros((16,), jnp.float32)
            return carry
        lax.fori_loop(0, CH, _zf, 0)

        def _zc(t, carry):
            pltpu.sync_copy(bufs[0], acc.at[pl.ds(s * STRIPE + t * CH, CH)])
            return carry
        lax.fori_loop(0, STRIPE // CH, _zc, 0)
        plsc.subcore_barrier()

        def _g_start(j, b):
            pltpu.async_copy(g_hbm.at[src_v.at[j]], bufs[b], gsems[b])

        def _g_wait(j, b):
            pltpu.make_async_copy(
                g_hbm.at[src_v.at[j]], bufs[b], gsems[b]).wait()

        def _s_start(j, b):
            pltpu.async_copy(bufs[b], acc.at[dst_v.at[j]], ssems[b], add=True)

        def _s_wait(j, b):
            pltpu.make_async_copy(bufs[b], acc.at[dst_v.at[j]], ssems[b]).wait()

        ntail = NCHUNK % NBUF
        nbody = NCHUNK - ntail

        for k in range(LOOK):
            _g_start(k, k)
        for j in range(NBUF):
            _g_wait(j, j)
            _s_start(j, j)
            bp = (j + LOOK) % NBUF
            if j >= NBUF - LOOK:
                _s_wait(j + LOOK - NBUF, bp)
            _g_start(j + LOOK, bp)

        def _grp(g, carry):
            for k in range(NBUF):
                j = g * NBUF + k
                _g_wait(j, k)
                _s_start(j, k)
                bp = (k + LOOK) % NBUF
                _s_wait(j + LOOK - NBUF, bp)
                _g_start(j + LOOK, bp)
            return carry
        lax.fori_loop(1, nbody // NBUF - 1, _grp, 0)

        for k in range(NBUF + ntail):
            j = nbody - NBUF + k
            _g_wait(j, j % NBUF)
            _s_start(j, j % NBUF)
            if j + LOOK < NCHUNK:
                bp = (j + LOOK) % NBUF
                _s_wait(j + LOOK - NBUF, bp)
                _g_start(j + LOOK, bp)
        for j in range(NCHUNK - NBUF, NCHUNK):
            _s_wait(j, j % NBUF)

        plsc.subcore_barrier()
        pltpu.sync_copy(acc.at[pl.ds(s * STRIPE, STRIPE)],
                        out_hbm.at[c, f, pl.ds(s * STRIPE, STRIPE)])


def _head_body(x_ref, klo_ref, khi_ref, dp_ref, dinv_ref, glo_ref, ghi_ref):
    deg = dp_ref[0] + dp_ref[1] + 1.0
    dinv = lax.rsqrt(deg)
    dinv_ref[...] = dinv
    mm = (((1,), (0,)), ((), ()))
    h_lo = lax.dot_general(x_ref[...], klo_ref[...], mm,
                           preferred_element_type=jnp.float32)
    h_hi = lax.dot_general(x_ref[...], khi_ref[...], mm,
                           preferred_element_type=jnp.float32)
    glo_ref[...] = h_lo * dinv
    ghi_ref[...] = h_hi * dinv


_head = pl.pallas_call(
    _head_body,
    grid=(NP2 // RBP,),
    in_specs=[pl.BlockSpec((RBP, 2 * D), lambda i: (i, 0)),
              pl.BlockSpec((2 * D, D), lambda i: (0, 0)),
              pl.BlockSpec((2 * D, D), lambda i: (0, 0)),
              pl.BlockSpec((NC, RBP, D), lambda i: (0, i, 0))],
    out_specs=[pl.BlockSpec((RBP, D), lambda i: (i, 0)),
               pl.BlockSpec((RBP, D), lambda i: (i, 0)),
               pl.BlockSpec((RBP, D), lambda i: (i, 0))],
    out_shape=[jax.ShapeDtypeStruct((NP2, D), jnp.float32),
               jax.ShapeDtypeStruct((NP2, D), jnp.float32),
               jax.ShapeDtypeStruct((NP2, D), jnp.float32)],
)


def _pre_acts(sp_ref, glo_ref, ghi_ref, dinv_ref, blo_ref, bhi_ref):
    t_lo = dinv_ref[...] * (sp_ref[0, 0] + sp_ref[1, 0] + glo_ref[...]) \
        + blo_ref[...]
    t_hi = dinv_ref[...] * (sp_ref[0, 1] + sp_ref[1, 1] + ghi_ref[...]) \
        + bhi_ref[...]
    return t_lo, t_hi


def _combine_body(sp_ref, glo_ref, ghi_ref, dinv_ref, blo_ref, bhi_ref,
                  all_ref, ahl_ref, alh_ref, ahh_ref, olo_ref, ohi_ref):
    t_lo, t_hi = _pre_acts(sp_ref, glo_ref, ghi_ref, dinv_ref,
                           blo_ref, bhi_ref)
    t_lo = jnp.maximum(t_lo, 0.0)
    t_hi = jnp.maximum(t_hi, 0.0)
    mm = (((1,), (0,)), ((), ()))
    h_lo = (lax.dot_general(t_lo, all_ref[...], mm,
                            preferred_element_type=jnp.float32)
            + lax.dot_general(t_hi, ahl_ref[...], mm,
                              preferred_element_type=jnp.float32))
    h_hi = (lax.dot_general(t_lo, alh_ref[...], mm,
                            preferred_element_type=jnp.float32)
            + lax.dot_general(t_hi, ahh_ref[...], mm,
                              preferred_element_type=jnp.float32))
    olo_ref[...] = dinv_ref[...] * h_lo
    ohi_ref[...] = dinv_ref[...] * h_hi


_wspec = pl.BlockSpec((D, D), lambda i: (0, 0))
_rspec = pl.BlockSpec((RBP, D), lambda i: (i, 0))

_combine = pl.pallas_call(
    _combine_body,
    grid=(NP2 // RBP,),
    in_specs=[pl.BlockSpec((NC, 2, RBP, D), lambda i: (0, 0, i, 0)),
              _rspec, _rspec, _rspec,
              pl.BlockSpec((1, D), lambda i: (0, 0)),
              pl.BlockSpec((1, D), lambda i: (0, 0)),
              _wspec, _wspec, _wspec, _wspec],
    out_specs=[_rspec, _rspec],
    out_shape=[jax.ShapeDtypeStruct((NP2, D), jnp.float32),
               jax.ShapeDtypeStruct((NP2, D), jnp.float32)],
)


def _final_body(sp_ref, glo_ref, ghi_ref, dinv_ref, blo_ref, bhi_ref,
                mlo_ref, mhi_ref, bs_ref, o_ref):
    t_lo, t_hi = _pre_acts(sp_ref, glo_ref, ghi_ref, dinv_ref,
                           blo_ref, bhi_ref)
    mm = (((1,), (0,)), ((), ()))
    sc = (lax.dot_general(t_lo, mlo_ref[...], mm,
                          preferred_element_type=jnp.float32)
          + lax.dot_general(t_hi, mhi_ref[...], mm,
                            preferred_element_type=jnp.float32))
    o_ref[...] = jax.nn.sigmoid(sc + bs_ref[0, 0])[:, :2]


_final = pl.pallas_call(
    _final_body,
    grid=(NP2 // RBP,),
    in_specs=[pl.BlockSpec((NC, 2, RBP, D), lambda i: (0, 0, i, 0)),
              _rspec, _rspec, _rspec,
              pl.BlockSpec((1, D), lambda i: (0, 0)),
              pl.BlockSpec((1, D), lambda i: (0, 0)),
              _wspec, _wspec,
              pl.BlockSpec((1, 1), lambda i: (0, 0))],
    out_specs=pl.BlockSpec((RBP, 2), lambda i: (i, 0)),
    out_shape=jax.ShapeDtypeStruct((NP2, 2), jnp.float32),
)


def _bdiag(m):
    a, b = m.shape
    z = jnp.zeros((2 * a, 2 * b), m.dtype)
    return z.at[:a, :b].set(m).at[a:, b:].set(m)


def _packed_weights(w):
    wt = w.T
    return (_bdiag(wt[:DH, :DH]), _bdiag(wt[DH:, :DH]),
            _bdiag(wt[:DH, DH:]), _bdiag(wt[DH:, DH:]))


def _packed_bias(b):
    return (jnp.tile(b[:DH], 2).reshape(1, D),
            jnp.tile(b[DH:], 2).reshape(1, D))


def kernel(x, edge_index, edge_attr, W1, b1, W2, b2, W3, b3, Ws, bs):
    del edge_attr
    ei4 = edge_index.reshape(2, NW, NCHUNK, CH)

    deg_packed = _deg_kernel(ei4)
    xp = x.reshape(N // 2, 2 * D)
    k1lo, k1hi = _bdiag(W1.T[:, :DH]), _bdiag(W1.T[:, DH:])
    dinv, g1lo, g1hi = _head(xp, k1lo, k1hi, deg_packed)

    out = None
    g_lo, g_hi = g1lo, g1hi
    for b, Wn in ((b1, W2), (b2, W3)):
        s = _scatter_kernel(g_lo.reshape(NPAD, DH), g_hi.reshape(NPAD, DH),
                            ei4)
        blo, bhi = _packed_bias(b)
        a_ll, a_hl, a_lh, a_hh = _packed_weights(Wn)
        g_lo, g_hi = _combine(s.reshape(NC, 2, NP2, D), g_lo, g_hi, dinv,
                              blo, bhi, a_ll, a_hl, a_lh, a_hh)

    s3 = _scatter_kernel(g_lo.reshape(NPAD, DH), g_hi.reshape(NPAD, DH), ei4)
    b3lo, b3hi = _packed_bias(b3)
    mlo = jnp.zeros((D, D), jnp.float32)
    mlo = mlo.at[:DH, 0].set(Ws[0, :DH]).at[DH:, 1].set(Ws[0, :DH])
    mhi = jnp.zeros((D, D), jnp.float32)
    mhi = mhi.at[:DH, 0].set(Ws[0, DH:]).at[DH:, 1].set(Ws[0, DH:])
    outp = _final(s3.reshape(NC, 2, NP2, D), g_lo, g_hi, dinv,
                  b3lo, b3hi, mlo, mhi, bs.reshape(1, 1))
    return outp.reshape(NPAD, 1)[:N]


# speedup vs baseline: 1.0040x; 1.0040x over previous
import functools

import jax
import jax.numpy as jnp
from jax import lax
from jax.experimental import pallas as pl
from jax.experimental.pallas import tpu as pltpu
from jax.experimental.pallas import tpu_sc as plsc

N = 10000
NPAD = 10240
E = 320000
D = 128
DH = D // 2
NC, NS = 2, 16
NW = NC * NS
CH = 80
EPT = E // NW
NCHUNK = EPT // CH
NBUF = 8
LOOK = 7
STRIPE = NPAD // NS
RB = 1024
NP2 = NPAD // 2
RBP = 512

_sc_mesh = plsc.VectorSubcoreMesh(
    core_axis_name="c", subcore_axis_name="s", num_cores=NC, num_subcores=NS)


@functools.partial(
    pl.kernel,
    out_type=jax.ShapeDtypeStruct((NC, NP2, D), jnp.float32),
    mesh=_sc_mesh,
    scratch_types=[
        pltpu.VMEM((EPT // 80, 80), jnp.int32),
        pltpu.VMEM((80,), jnp.float32),
        pltpu.VMEM((STRIPE,), jnp.float32),
        pltpu.VMEM((STRIPE // 2, D), jnp.float32),
        pltpu.VMEM_SHARED((NPAD,), jnp.float32),
    ],
    compiler_params=pltpu.CompilerParams(needs_layout_passes=False),
)
def _deg_kernel(ei_hbm, out_hbm, idx_v, ones_v, zero_v, bb, acc):
    c = lax.axis_index("c")
    s = lax.axis_index("s")
    wid = c * NS + s
    nch = EPT // 80

    for k in range(80 // 16):
        ones_v[pl.ds(k * 16, 16)] = jnp.ones((16,), jnp.float32)

    def _zf(i, carry):
        zero_v[pl.ds(i * 16, 16)] = jnp.zeros((16,), jnp.float32)
        return carry
    lax.fori_loop(0, STRIPE // 16, _zf, 0)

    pltpu.sync_copy(ei_hbm.at[1, wid], idx_v)
    pltpu.sync_copy(zero_v, acc.at[pl.ds(s * STRIPE, STRIPE)])
    plsc.subcore_barrier()

    def _body(j, carry):
        pltpu.sync_copy(ones_v, acc.at[idx_v.at[j]], add=True)
        return carry
    lax.fori_loop(0, nch, _body, 0)

    plsc.subcore_barrier()
    pltpu.sync_copy(acc.at[pl.ds(s * STRIPE, STRIPE)], zero_v)

    def _bc(r, carry):
        v0 = plsc.load_gather(zero_v, [jnp.full((16,), 2 * r, jnp.int32)])
        v1 = plsc.load_gather(zero_v, [jnp.full((16,), 2 * r + 1, jnp.int32)])
        for q in range(DH // 16):
            bb[r, pl.ds(q * 16, 16)] = v0
        for q in range(DH // 16):
            bb[r, pl.ds(DH + q * 16, 16)] = v1
        return carry
    lax.fori_loop(0, STRIPE // 2, _bc, 0)
    pltpu.sync_copy(bb, out_hbm.at[c, pl.ds(s * (STRIPE // 2), STRIPE // 2)])


@functools.partial(
    pl.kernel,
    out_type=jax.ShapeDtypeStruct((NC, 2, NPAD, DH), jnp.float32),
    mesh=_sc_mesh,
    scratch_types=[
        pltpu.VMEM((NCHUNK, CH), jnp.int32),
        pltpu.VMEM((NCHUNK, CH), jnp.int32),
        [pltpu.VMEM((CH, DH), jnp.float32) for _ in range(NBUF)],
        pltpu.VMEM_SHARED((NPAD, DH), jnp.float32),
        [pltpu.SemaphoreType.DMA for _ in range(NBUF)],
        [pltpu.SemaphoreType.DMA for _ in range(NBUF)],
    ],
    compiler_params=pltpu.CompilerParams(use_tc_tiling_on_sc=False),
)
def _scatter_kernel(g_lo_hbm, g_hi_hbm, ei_hbm, out_hbm,
                    src_v, dst_v, bufs, acc, gsems, ssems):
    c = lax.axis_index("c")
    s = lax.axis_index("s")
    wid = c * NS + s

    pltpu.sync_copy(ei_hbm.at[0, wid], src_v)
    pltpu.sync_copy(ei_hbm.at[1, wid], dst_v)

    for f, g_hbm in ((0, g_lo_hbm), (1, g_hi_hbm)):
        def _zf(i, carry):
            for k in range(DH // 16):
                bufs[0][i, pl.ds(k * 16, 16)] = jnp.zeros((16,), jnp.float32)
            return carry
        lax.fori_loop(0, CH, _zf, 0)

        def _zc(t, carry):
            pltpu.sync_copy(bufs[0], acc.at[pl.ds(s * STRIPE + t * CH, CH)])
            return carry
        lax.fori_loop(0, STRIPE // CH, _zc, 0)
        plsc.subcore_barrier()

        def _g_start(j, b):
            pltpu.async_copy(g_hbm.at[src_v.at[j]], bufs[b], gsems[b])

        def _g_wait(j, b):
            pltpu.make_async_copy(
                g_hbm.at[src_v.at[j]], bufs[b], gsems[b]).wait()

        def _s_start(j, b):
            pltpu.async_copy(bufs[b], acc.at[dst_v.at[j]], ssems[b], add=True)

        def _s_wait(j, b):
            pltpu.make_async_copy(bufs[b], acc.at[dst_v.at[j]], ssems[b]).wait()

        ntail = NCHUNK % NBUF
        nbody = NCHUNK - ntail

        for k in range(LOOK):
            _g_start(k, k)
        for j in range(NBUF):
            _g_wait(j, j)
            _s_start(j, j)
            bp = (j + LOOK) % NBUF
            if j >= NBUF - LOOK:
                _s_wait(j + LOOK - NBUF, bp)
            _g_start(j + LOOK, bp)

        def _grp(g, carry):
            for k in range(NBUF):
                j = g * NBUF + k
                _g_wait(j, k)
                _s_start(j, k)
                bp = (k + LOOK) % NBUF
                _s_wait(j + LOOK - NBUF, bp)
                _g_start(j + LOOK, bp)
            return carry
        lax.fori_loop(1, nbody // NBUF - 1, _grp, 0)

        for k in range(NBUF + ntail):
            j = nbody - NBUF + k
            _g_wait(j, j % NBUF)
            _s_start(j, j % NBUF)
            if j + LOOK < NCHUNK:
                bp = (j + LOOK) % NBUF
                _s_wait(j + LOOK - NBUF, bp)
                _g_start(j + LOOK, bp)
        for j in range(NCHUNK - NBUF, NCHUNK):
            _s_wait(j, j % NBUF)

        plsc.subcore_barrier()
        pltpu.sync_copy(acc.at[pl.ds(s * STRIPE, STRIPE)],
                        out_hbm.at[c, f, pl.ds(s * STRIPE, STRIPE)])


def _head_body(x_ref, klo_ref, khi_ref, dp_ref, dinv_ref, glo_ref, ghi_ref):
    deg = dp_ref[0] + dp_ref[1] + 1.0
    dinv = lax.rsqrt(deg)
    dinv_ref[...] = dinv
    mm = (((1,), (0,)), ((), ()))
    h_lo = lax.dot_general(x_ref[...], klo_ref[...], mm,
                           preferred_element_type=jnp.float32)
    h_hi = lax.dot_general(x_ref[...], khi_ref[...], mm,
                           preferred_element_type=jnp.float32)
    glo_ref[...] = h_lo * dinv
    ghi_ref[...] = h_hi * dinv


_head = pl.pallas_call(
    _head_body,
    grid=(NP2 // RBP,),
    in_specs=[pl.BlockSpec((RBP, 2 * D), lambda i: (i, 0)),
              pl.BlockSpec((2 * D, D), lambda i: (0, 0)),
              pl.BlockSpec((2 * D, D), lambda i: (0, 0)),
              pl.BlockSpec((NC, RBP, D), lambda i: (0, i, 0))],
    out_specs=[pl.BlockSpec((RBP, D), lambda i: (i, 0)),
               pl.BlockSpec((RBP, D), lambda i: (i, 0)),
               pl.BlockSpec((RBP, D), lambda i: (i, 0))],
    out_shape=[jax.ShapeDtypeStruct((NP2, D), jnp.float32),
               jax.ShapeDtypeStruct((NP2, D), jnp.float32),
               jax.ShapeDtypeStruct((NP2, D), jnp.float32)],
)


def _pre_acts(sp_ref, glo_ref, ghi_ref, dinv_ref, blo_ref, bhi_ref):
    t_lo = dinv_ref[...] * (sp_ref[0, 0] + sp_ref[1, 0] + glo_ref[...]) \
        + blo_ref[...]
    t_hi = dinv_ref[...] * (sp_ref[0, 1] + sp_ref[1, 1] + ghi_ref[...]) \
        + bhi_ref[...]
    return t_lo, t_hi


def _combine_body(sp_ref, glo_ref, ghi_ref, dinv_ref, blo_ref, bhi_ref,
                  all_ref, ahl_ref, alh_ref, ahh_ref, olo_ref, ohi_ref):
    t_lo, t_hi = _pre_acts(sp_ref, glo_ref, ghi_ref, dinv_ref,
                           blo_ref, bhi_ref)
    t_lo = jnp.maximum(t_lo, 0.0)
    t_hi = jnp.maximum(t_hi, 0.0)
    mm = (((1,), (0,)), ((), ()))
    h_lo = (lax.dot_general(t_lo, all_ref[...], mm,
                            preferred_element_type=jnp.float32)
            + lax.dot_general(t_hi, ahl_ref[...], mm,
                              preferred_element_type=jnp.float32))
    h_hi = (lax.dot_general(t_lo, alh_ref[...], mm,
                            preferred_element_type=jnp.float32)
            + lax.dot_general(t_hi, ahh_ref[...], mm,
                              preferred_element_type=jnp.float32))
    olo_ref[...] = dinv_ref[...] * h_lo
    ohi_ref[...] = dinv_ref[...] * h_hi


_wspec = pl.BlockSpec((D, D), lambda i: (0, 0))
_rspec = pl.BlockSpec((RBP, D), lambda i: (i, 0))

_combine = pl.pallas_call(
    _combine_body,
    grid=(NP2 // RBP,),
    in_specs=[pl.BlockSpec((NC, 2, RBP, D), lambda i: (0, 0, i, 0)),
              _rspec, _rspec, _rspec,
              pl.BlockSpec((1, D), lambda i: (0, 0)),
              pl.BlockSpec((1, D), lambda i: (0, 0)),
              _wspec, _wspec, _wspec, _wspec],
    out_specs=[_rspec, _rspec],
    out_shape=[jax.ShapeDtypeStruct((NP2, D), jnp.float32),
               jax.ShapeDtypeStruct((NP2, D), jnp.float32)],
)


def _final_body(sp_ref, glo_ref, ghi_ref, dinv_ref, blo_ref, bhi_ref,
                mlo_ref, mhi_ref, bs_ref, o_ref):
    t_lo, t_hi = _pre_acts(sp_ref, glo_ref, ghi_ref, dinv_ref,
                           blo_ref, bhi_ref)
    mm = (((1,), (0,)), ((), ()))
    sc = (lax.dot_general(t_lo, mlo_ref[...], mm,
                          preferred_element_type=jnp.float32)
          + lax.dot_general(t_hi, mhi_ref[...], mm,
                            preferred_element_type=jnp.float32))
    o_ref[...] = jax.nn.sigmoid(sc + bs_ref[0, 0])[:, :2]


_final = pl.pallas_call(
    _final_body,
    grid=(NP2 // RBP,),
    in_specs=[pl.BlockSpec((NC, 2, RBP, D), lambda i: (0, 0, i, 0)),
              _rspec, _rspec, _rspec,
              pl.BlockSpec((1, D), lambda i: (0, 0)),
              pl.BlockSpec((1, D), lambda i: (0, 0)),
              _wspec, _wspec,
              pl.BlockSpec((1, 1), lambda i: (0, 0))],
    out_specs=pl.BlockSpec((RBP, 2), lambda i: (i, 0)),
    out_shape=jax.ShapeDtypeStruct((NP2, 2), jnp.float32),
)


def _bdiag(m):
    a, b = m.shape
    z = jnp.zeros((2 * a, 2 * b), m.dtype)
    return z.at[:a, :b].set(m).at[a:, b:].set(m)


def _packed_weights(w):
    wt = w.T
    return (_bdiag(wt[:DH, :DH]), _bdiag(wt[DH:, :DH]),
            _bdiag(wt[:DH, DH:]), _bdiag(wt[DH:, DH:]))


def _packed_bias(b):
    return (jnp.tile(b[:DH], 2).reshape(1, D),
            jnp.tile(b[DH:], 2).reshape(1, D))


def kernel(x, edge_index, edge_attr, W1, b1, W2, b2, W3, b3, Ws, bs):
    del edge_attr
    ei4 = edge_index.reshape(2, NW, NCHUNK, CH)

    deg_packed = _deg_kernel(ei4)
    xp = x.reshape(N // 2, 2 * D)
    k1lo, k1hi = _bdiag(W1.T[:, :DH]), _bdiag(W1.T[:, DH:])
    dinv, g1lo, g1hi = _head(xp, k1lo, k1hi, deg_packed)

    out = None
    g_lo, g_hi = g1lo, g1hi
    for b, Wn in ((b1, W2), (b2, W3)):
        s = _scatter_kernel(g_lo.reshape(NPAD, DH), g_hi.reshape(NPAD, DH),
                            ei4)
        blo, bhi = _packed_bias(b)
        a_ll, a_hl, a_lh, a_hh = _packed_weights(Wn)
        g_lo, g_hi = _combine(s.reshape(NC, 2, NP2, D), g_lo, g_hi, dinv,
                              blo, bhi, a_ll, a_hl, a_lh, a_hh)

    s3 = _scatter_kernel(g_lo.reshape(NPAD, DH), g_hi.reshape(NPAD, DH), ei4)
    b3lo, b3hi = _packed_bias(b3)
    mlo = jnp.zeros((D, D), jnp.float32)
    mlo = mlo.at[:DH, 0].set(Ws[0, :DH]).at[DH:, 1].set(Ws[0, :DH])
    mhi = jnp.zeros((D, D), jnp.float32)
    mhi = mhi.at[:DH, 0].set(Ws[0, DH:]).at[DH:, 1].set(Ws[0, DH:])
    outp = _final(s3.reshape(NC, 2, NP2, D), g_lo, g_hi, dinv,
                  b3lo, b3hi, mlo, mhi, bs.reshape(1, 1))
    return outp.reshape(NPAD, 1)[:N]
